# baseline (speedup 1.0000x reference)
import functools

import jax
import jax.numpy as jnp
from jax import lax
from jax.experimental import pallas as pl
from jax.experimental.pallas import tpu as pltpu
from jax.experimental.pallas import tpu_sc as plsc

NC = 2
NS = 16
NW = NC * NS
B = 128
NBUF = 8
GAHEAD = 4


@functools.cache
def _build(N, E, D, H, C):
    assert H == 16, "row width must match the 64B DMA granule"
    Hp = 16
    NP = -(-(N + 16) // 128) * 128
    CHUNK = NP // NS
    S = -(-E // (NW * B))
    S = max(S + (S % 2), 2 * NBUF)
    EP = S * NW * B

    mesh = plsc.VectorSubcoreMesh(core_axis_name="c", subcore_axis_name="s")
    sc_params = pltpu.CompilerParams(use_tc_tiling_on_sc=False)

    @functools.partial(
        pl.kernel,
        out_type=jax.ShapeDtypeStruct((NC, NP), jnp.float32),
        mesh=mesh,
        compiler_params=sc_params,
        scratch_types=[
            pltpu.VMEM((S, B), jnp.int32),
            pltpu.VMEM((B,), jnp.float32),
            pltpu.VMEM_SHARED((NP,), jnp.float32),
            pltpu.SemaphoreType.DMA((NBUF,)),
        ],
    )
    def deg_kernel(dst_hbm, zn_hbm, out_hbm, dst_v, ones_v, dacc_sh, ssem):
        cid = lax.axis_index("c")
        sid = lax.axis_index("s")
        wid = sid * NC + cid
        pltpu.sync_copy(dst_hbm.at[wid], dst_v)
        for i in range(B // 16):
            ones_v[pl.ds(i * 16, 16)] = jnp.ones((16,), jnp.float32)
        pltpu.sync_copy(zn_hbm.at[pl.ds(sid * CHUNK, CHUNK)],
                        dacc_sh.at[pl.ds(sid * CHUNK, CHUNK)])
        plsc.subcore_barrier()

        def body(j, carry):
            b = lax.rem(j, NBUF)

            @pl.when(j >= NBUF)
            def _wait_prev():
                pltpu.make_async_copy(
                    ones_v, dacc_sh.at[dst_v.at[j]], ssem.at[b]).wait()

            pltpu.async_copy(ones_v, dacc_sh.at[dst_v.at[j]], ssem.at[b],
                             add=True)
            return carry

        lax.fori_loop(0, S, body, 0)
        for b in range(NBUF):
            pltpu.make_async_copy(
                ones_v, dacc_sh.at[dst_v.at[0]], ssem.at[b]).wait()
        plsc.subcore_barrier()
        pltpu.sync_copy(dacc_sh.at[pl.ds(sid * CHUNK, CHUNK)],
                        out_hbm.at[cid, pl.ds(sid * CHUNK, CHUNK)])

    @functools.partial(
        pl.kernel,
        out_type=jax.ShapeDtypeStruct((NC, NP, Hp), jnp.float32),
        mesh=mesh,
        compiler_params=sc_params,
        scratch_types=[
            pltpu.VMEM((S, B), jnp.int32),
            pltpu.VMEM((S, B), jnp.int32),
            pltpu.VMEM((NBUF, B, Hp), jnp.float32),
            pltpu.VMEM_SHARED((NP, Hp), jnp.float32),
            pltpu.SemaphoreType.DMA((NBUF,)),
            pltpu.SemaphoreType.DMA((NBUF,)),
        ],
    )
    def edge_kernel(src_hbm, dst_hbm, y_hbm, z_hbm, out_hbm,
                    src_v, dst_v, rows_v, acc_sh, gsem, ssem):
        cid = lax.axis_index("c")
        sid = lax.axis_index("s")
        wid = sid * NC + cid
        pltpu.sync_copy(src_hbm.at[wid], src_v)
        pltpu.sync_copy(dst_hbm.at[wid], dst_v)
        pltpu.sync_copy(z_hbm.at[pl.ds(sid * CHUNK, CHUNK)],
                        acc_sh.at[pl.ds(sid * CHUNK, CHUNK)])
        plsc.subcore_barrier()

        def body(j, carry):
            @pl.when(j < S)
            def _gather():
                b = lax.rem(j, NBUF)

                @pl.when(j >= NBUF)
                def _wait_scatter():
                    pltpu.make_async_copy(
                        rows_v.at[b], acc_sh.at[dst_v.at[0]], ssem.at[b]).wait()

                pltpu.async_copy(
                    y_hbm.at[src_v.at[j]], rows_v.at[b], gsem.at[b])

            jj = j - GAHEAD

            @pl.when(jj >= 0)
            def _scatter():
                bb = lax.rem(jj, NBUF)
                pltpu.make_async_copy(
                    y_hbm.at[src_v.at[jj]], rows_v.at[bb], gsem.at[bb]).wait()
                pltpu.async_copy(
                    rows_v.at[bb], acc_sh.at[dst_v.at[jj]], ssem.at[bb],
                    add=True)

            return carry

        lax.fori_loop(0, S + GAHEAD, body, 0)
        for b in range(NBUF):
            pltpu.make_async_copy(
                rows_v.at[b], acc_sh.at[dst_v.at[0]], ssem.at[b]).wait()
        plsc.subcore_barrier()
        pltpu.sync_copy(acc_sh.at[pl.ds(sid * CHUNK, CHUNK)],
                        out_hbm.at[cid, pl.ds(sid * CHUNK, CHUNK)])

    def mm_body(x_ref, w_ref, o_ref):
        o_ref[...] = jnp.dot(x_ref[...], w_ref[...],
                             preferred_element_type=jnp.float32)

    mm1 = pl.pallas_call(
        mm_body, out_shape=jax.ShapeDtypeStruct((N, Hp), jnp.float32))

    def scale_body(xw_ref, d0_ref, d1_ref, y_ref, dinv_ref):
        dinv = lax.rsqrt(d0_ref[...] + d1_ref[...] + 1.0)
        y_ref[...] = xw_ref[...] * dinv
        dinv_ref[...] = dinv

    scale = pl.pallas_call(
        scale_body,
        out_shape=(jax.ShapeDtypeStruct((N, Hp), jnp.float32),
                   jax.ShapeDtypeStruct((N, 1), jnp.float32)))

    def mid_body(a0_ref, a1_ref, y1_ref, dinv_ref, b1_ref, w2_ref, y2_ref):
        dinv = dinv_ref[...]
        h = dinv * (a0_ref[...] + a1_ref[...] + y1_ref[...]) + b1_ref[...]
        h = jnp.maximum(h, 0.0)
        y2_ref[...] = jnp.dot(h, w2_ref[...],
                              preferred_element_type=jnp.float32) * dinv

    mid = pl.pallas_call(
        mid_body, out_shape=jax.ShapeDtypeStruct((N, Hp), jnp.float32))

    def fin_body(a0_ref, a1_ref, y2_ref, dinv_ref, b2_ref, o_ref):
        o = dinv_ref[...] * (a0_ref[...] + a1_ref[...] + y2_ref[...]) + b2_ref[...]
        col = lax.broadcasted_iota(jnp.int32, o.shape, 1)
        om = jnp.where(col < C, o, -jnp.inf)
        m = jnp.max(om, axis=1, keepdims=True)
        s = jnp.sum(jnp.where(col < C, jnp.exp(o - m), 0.0), axis=1,
                    keepdims=True)
        o_ref[...] = o - (m + jnp.log(s))

    fin = pl.pallas_call(
        fin_body, out_shape=jax.ShapeDtypeStruct((N, Hp), jnp.float32))

    return deg_kernel, edge_kernel, mm1, scale, mid, fin, NP, S, EP


def kernel(x, edge_index, W1, b1, W2, b2):
    N, D = x.shape
    E = edge_index.shape[1]
    H = W1.shape[1]
    C = W2.shape[1]
    Hp = 16
    (deg_kernel, edge_kernel, mm1, scale, mid, fin,
     NP, S, EP) = _build(N, E, D, H, C)

    src = edge_index[0].astype(jnp.int32)
    dst = edge_index[1].astype(jnp.int32)
    pad = EP - E
    lanes = jnp.arange(pad, dtype=jnp.int32) % 16
    srcp = jnp.concatenate([src, lanes]).reshape(NW, S, B)
    dstp = jnp.concatenate([dst, N + lanes]).reshape(NW, S, B)

    zn = jnp.zeros((NP,), jnp.float32)
    zr = jnp.zeros((NP, Hp), jnp.float32)
    W2p = jnp.concatenate(
        [W2, jnp.zeros((H, Hp - C), jnp.float32)], axis=1)
    b1r = b1.reshape(1, H)
    b2p = jnp.concatenate([b2, jnp.zeros((Hp - C,), jnp.float32)]).reshape(1, Hp)

    dpart = deg_kernel(dstp, zn)
    xw = mm1(x, W1)
    y1, dinv = scale(xw, dpart[0, :N, None], dpart[1, :N, None])
    a = edge_kernel(srcp, dstp, y1, zr)
    y2 = mid(a[0, :N], a[1, :N], y1, dinv, b1r, W2p)
    a2 = edge_kernel(srcp, dstp, y2, zr)
    out = fin(a2[0, :N], a2[1, :N], y2, dinv, b2p)
    return out[:, :C]


# speedup vs baseline: 1.1526x; 1.1526x over previous
import functools

import jax
import jax.numpy as jnp
from jax import lax
from jax.experimental import pallas as pl
from jax.experimental.pallas import tpu as pltpu
from jax.experimental.pallas import tpu_sc as plsc

NC = 2
NS = 16
NW = NC * NS
B = 128
NBUF = 8
GAHEAD = 4


@functools.cache
def _build(N, E, D, H, C):
    assert H == 16, "row width must match the 64B DMA granule"
    Hp = 16
    NP = -(-(N + 16) // 256) * 256
    CHUNK = NP // NS
    S = -(-E // (NW * B))
    S = max(S + (S % 2), NBUF)
    EP = S * NW * B

    mesh = plsc.VectorSubcoreMesh(core_axis_name="c", subcore_axis_name="s")
    sc_params = pltpu.CompilerParams(use_tc_tiling_on_sc=False)

    @functools.partial(
        pl.kernel,
        out_type=jax.ShapeDtypeStruct((NC, NP), jnp.float32),
        mesh=mesh,
        compiler_params=sc_params,
        scratch_types=[
            pltpu.VMEM((S, B), jnp.int32),
            pltpu.VMEM((B,), jnp.float32),
            pltpu.VMEM((CHUNK,), jnp.float32),
            pltpu.VMEM_SHARED((NP,), jnp.float32),
            pltpu.SemaphoreType.DMA((NBUF,)),
        ],
    )
    def launch_h(dst_hbm, deg_out, dst_v, ones_v, z_v, deg_sh, ssem):
        cid = lax.axis_index("c")
        sid = lax.axis_index("s")
        lo = sid * CHUNK
        pltpu.sync_copy(dst_hbm.at[2 * sid + cid], dst_v)
        for i in range(B // 16):
            ones_v[pl.ds(i * 16, 16)] = jnp.ones((16,), jnp.float32)

        def zbody(i, carry):
            z_v[pl.ds(i * 16, 16)] = jnp.zeros((16,), jnp.float32)
            return carry

        lax.fori_loop(0, CHUNK // 16, zbody, 0)
        pltpu.sync_copy(z_v, deg_sh.at[pl.ds(lo, CHUNK)])
        plsc.subcore_barrier()

        def dbody(j, carry):
            b = lax.rem(j, NBUF)

            @pl.when(j >= NBUF)
            def _wait_prev():
                pltpu.make_async_copy(
                    ones_v, deg_sh.at[dst_v.at[0]], ssem.at[b]).wait()

            pltpu.async_copy(ones_v, deg_sh.at[dst_v.at[j]], ssem.at[b],
                             add=True)
            return carry

        lax.fori_loop(0, S, dbody, 0)
        for b in range(NBUF):
            pltpu.make_async_copy(
                ones_v, deg_sh.at[dst_v.at[0]], ssem.at[b]).wait()
        plsc.subcore_barrier()
        pltpu.sync_copy(deg_sh.at[pl.ds(lo, CHUNK)],
                        deg_out.at[cid, pl.ds(lo, CHUNK)])

    @functools.partial(
        pl.kernel,
        out_type=jax.ShapeDtypeStruct((NC, NP, Hp), jnp.float32),
        mesh=mesh,
        compiler_params=sc_params,
        scratch_types=[
            pltpu.VMEM((S, B), jnp.int32),
            pltpu.VMEM((S, B), jnp.int32),
            pltpu.VMEM((NBUF, B, Hp), jnp.float32),
            pltpu.VMEM_SHARED((NP, Hp), jnp.float32),
            pltpu.SemaphoreType.DMA((NBUF,)),
            pltpu.SemaphoreType.DMA((NBUF,)),
        ],
    )
    def launch_e(src_hbm, dst_hbm, y_hbm, z_hbm,
                 a_out, src_v, dst_v, rows_v, acc_sh, gsem, ssem):
        cid = lax.axis_index("c")
        sid = lax.axis_index("s")
        lo = sid * CHUNK
        pltpu.sync_copy(src_hbm.at[2 * sid + cid], src_v)
        pltpu.sync_copy(dst_hbm.at[2 * sid + cid], dst_v)
        pltpu.sync_copy(z_hbm.at[pl.ds(lo, CHUNK)],
                        acc_sh.at[pl.ds(lo, CHUNK)])
        plsc.subcore_barrier()

        def body(j, carry):
            @pl.when(j < S)
            def _gather():
                b = lax.rem(j, NBUF)

                @pl.when(j >= NBUF)
                def _wait_scatter():
                    pltpu.make_async_copy(
                        rows_v.at[b], acc_sh.at[src_v.at[0]], ssem.at[b]).wait()

                pltpu.async_copy(y_hbm.at[src_v.at[j]], rows_v.at[b],
                                 gsem.at[b])

            jj = j - GAHEAD

            @pl.when(jj >= 0)
            def _scatter():
                bb = lax.rem(jj, NBUF)
                pltpu.make_async_copy(
                    y_hbm.at[src_v.at[jj]], rows_v.at[bb], gsem.at[bb]).wait()
                pltpu.async_copy(
                    rows_v.at[bb], acc_sh.at[dst_v.at[jj]], ssem.at[bb],
                    add=True)

            return carry

        lax.fori_loop(0, S + GAHEAD, body, 0)
        for b in range(NBUF):
            pltpu.make_async_copy(
                rows_v.at[b], acc_sh.at[src_v.at[0]], ssem.at[b]).wait()
        plsc.subcore_barrier()
        pltpu.sync_copy(acc_sh.at[pl.ds(lo, CHUNK)],
                        a_out.at[cid, pl.ds(lo, CHUNK)])

    def mm_body(x_ref, w_ref, o_ref):
        o_ref[pl.ds(0, N), :] = jnp.dot(x_ref[...], w_ref[...],
                                        preferred_element_type=jnp.float32)
        o_ref[pl.ds(N, NP - N), :] = jnp.zeros((NP - N, Hp), jnp.float32)

    mm1 = pl.pallas_call(
        mm_body, out_shape=jax.ShapeDtypeStruct((NP, Hp), jnp.float32))

    def prep_body(deg_ref, xw_ref, y_ref, dinv_ref):
        dt = deg_ref[0, :] + deg_ref[1, :]
        dinv = lax.rsqrt(dt + 1.0)[:, None]
        dinv_ref[...] = dinv
        y_ref[...] = xw_ref[...] * dinv

    prep = pl.pallas_call(
        prep_body,
        out_shape=(jax.ShapeDtypeStruct((NP, Hp), jnp.float32),
                   jax.ShapeDtypeStruct((NP, 1), jnp.float32)))

    def g_body(a_ref, y_ref, dinv_ref, b1_ref, o_ref):
        s = a_ref[0] + a_ref[1] + y_ref[...]
        di = dinv_ref[...]
        o_ref[...] = di * jnp.maximum(s * di + b1_ref[...], 0.0)

    gstage = pl.pallas_call(
        g_body, out_shape=jax.ShapeDtypeStruct((NP, Hp), jnp.float32))

    def fin_body(c0_ref, c1_ref, g_ref, dinv_ref, w2_ref, b2_ref, o_ref):
        t = dinv_ref[...] * (c0_ref[...] + c1_ref[...] + g_ref[...])
        o = jnp.dot(t, w2_ref[...],
                    preferred_element_type=jnp.float32) + b2_ref[...]
        col = lax.broadcasted_iota(jnp.int32, o.shape, 1)
        om = jnp.where(col < C, o, -jnp.inf)
        m = jnp.max(om, axis=1, keepdims=True)
        s = jnp.sum(jnp.where(col < C, jnp.exp(o - m), 0.0), axis=1,
                    keepdims=True)
        o_ref[...] = o - (m + jnp.log(s))

    fin = pl.pallas_call(
        fin_body, out_shape=jax.ShapeDtypeStruct((N, Hp), jnp.float32))

    return launch_h, launch_e, mm1, prep, gstage, fin, NP, S, EP


def kernel(x, edge_index, W1, b1, W2, b2):
    N, D = x.shape
    E = edge_index.shape[1]
    H = W1.shape[1]
    C = W2.shape[1]
    Hp = 16
    (launch_h, launch_e, mm1, prep, gstage, fin,
     NP, S, EP) = _build(N, E, D, H, C)

    src = edge_index[0].astype(jnp.int32)
    dst = edge_index[1].astype(jnp.int32)
    pad = EP - E
    lanes = jnp.arange(pad, dtype=jnp.int32) % 16
    srcp = jnp.concatenate([src, lanes]).reshape(NW, S, B)
    dstp = jnp.concatenate([dst, N + lanes]).reshape(NW, S, B)

    zr = jnp.zeros((NP, Hp), jnp.float32)
    W2p = jnp.concatenate(
        [W2, jnp.zeros((H, Hp - C), jnp.float32)], axis=1)
    b1r = b1.reshape(1, Hp)
    b2p = jnp.concatenate([b2, jnp.zeros((Hp - C,), jnp.float32)]).reshape(1, Hp)

    deg = launch_h(dstp)
    xw = mm1(x, W1)
    y1, dinv = prep(deg, xw)
    a = launch_e(srcp, dstp, y1, zr)
    g = gstage(a, y1, dinv, b1r)
    c = launch_e(srcp, dstp, g, zr)
    out = fin(c[0, :N], c[1, :N], g[:N], dinv[:N], W2p, b2p)
    return out[:, :C]


# speedup vs baseline: 1.1865x; 1.0294x over previous
import functools

import jax
import jax.numpy as jnp
from jax import lax
from jax.experimental import pallas as pl
from jax.experimental.pallas import tpu as pltpu
from jax.experimental.pallas import tpu_sc as plsc

NC = 2
NS = 16
NW = NC * NS
B = 128
NBUF = 8
GAHEAD = 4


@functools.cache
def _build(N, E, D, H, C):
    assert H == 16, "row width must match the 64B DMA granule"
    Hp = 16
    NP = -(-(N + 16) // 256) * 256
    CHUNK = NP // NS
    S = -(-E // (NW * B))
    S = max(S + (S % 2), NBUF)
    EP = S * NW * B

    mesh = plsc.VectorSubcoreMesh(core_axis_name="c", subcore_axis_name="s")
    sc_params = pltpu.CompilerParams(use_tc_tiling_on_sc=False)

    @functools.partial(
        pl.kernel,
        out_type=jax.ShapeDtypeStruct((NC, NP), jnp.float32),
        mesh=mesh,
        compiler_params=sc_params,
        scratch_types=[
            pltpu.VMEM((S, B), jnp.int32),
            pltpu.VMEM((B,), jnp.float32),
            pltpu.VMEM((CHUNK,), jnp.float32),
            pltpu.VMEM_SHARED((NP,), jnp.float32),
            pltpu.SemaphoreType.DMA((NBUF,)),
        ],
    )
    def launch_h(dst_hbm, deg_out, dst_v, ones_v, z_v, deg_sh, ssem):
        cid = lax.axis_index("c")
        sid = lax.axis_index("s")
        lo = sid * CHUNK
        pltpu.sync_copy(dst_hbm.at[2 * sid + cid], dst_v)
        for i in range(B // 16):
            ones_v[pl.ds(i * 16, 16)] = jnp.ones((16,), jnp.float32)

        def zbody(i, carry):
            z_v[pl.ds(i * 16, 16)] = jnp.zeros((16,), jnp.float32)
            return carry

        lax.fori_loop(0, CHUNK // 16, zbody, 0)
        pltpu.sync_copy(z_v, deg_sh.at[pl.ds(lo, CHUNK)])
        plsc.subcore_barrier()

        def dbody(j, carry):
            b = lax.rem(j, NBUF)

            @pl.when(j >= NBUF)
            def _wait_prev():
                pltpu.make_async_copy(
                    ones_v, deg_sh.at[dst_v.at[0]], ssem.at[b]).wait()

            pltpu.async_copy(ones_v, deg_sh.at[dst_v.at[j]], ssem.at[b],
                             add=True)
            return carry

        lax.fori_loop(0, S, dbody, 0)
        for b in range(NBUF):
            pltpu.make_async_copy(
                ones_v, deg_sh.at[dst_v.at[0]], ssem.at[b]).wait()
        plsc.subcore_barrier()
        pltpu.sync_copy(deg_sh.at[pl.ds(lo, CHUNK)],
                        deg_out.at[cid, pl.ds(lo, CHUNK)])

    @functools.partial(
        pl.kernel,
        out_type=jax.ShapeDtypeStruct((NC, NP, Hp), jnp.float32),
        mesh=mesh,
        compiler_params=sc_params,
        scratch_types=[
            pltpu.VMEM((S, B), jnp.int32),
            pltpu.VMEM((S, B), jnp.int32),
            pltpu.VMEM((NBUF, B, Hp), jnp.float32),
            pltpu.VMEM_SHARED((NP, Hp), jnp.float32),
            pltpu.SemaphoreType.DMA((NBUF,)),
            pltpu.SemaphoreType.DMA((NBUF,)),
        ],
    )
    def launch_e(src_hbm, dst_hbm, y_hbm, z_hbm,
                 a_out, src_v, dst_v, rows_v, acc_sh, gsem, ssem):
        cid = lax.axis_index("c")
        sid = lax.axis_index("s")
        lo = sid * CHUNK
        pltpu.sync_copy(src_hbm.at[2 * sid + cid], src_v)
        pltpu.sync_copy(dst_hbm.at[2 * sid + cid], dst_v)
        pltpu.sync_copy(z_hbm.at[pl.ds(lo, CHUNK)],
                        acc_sh.at[pl.ds(lo, CHUNK)])
        plsc.subcore_barrier()

        def body(j, carry):
            @pl.when(j < S)
            def _gather():
                b = lax.rem(j, NBUF)

                @pl.when(j >= NBUF)
                def _wait_scatter():
                    pltpu.make_async_copy(
                        rows_v.at[b], acc_sh.at[src_v.at[0]], ssem.at[b]).wait()

                pltpu.async_copy(y_hbm.at[src_v.at[j]], rows_v.at[b],
                                 gsem.at[b])

            jj = j - GAHEAD

            @pl.when(jj >= 0)
            def _scatter():
                bb = lax.rem(jj, NBUF)
                pltpu.make_async_copy(
                    y_hbm.at[src_v.at[jj]], rows_v.at[bb], gsem.at[bb]).wait()
                pltpu.async_copy(
                    rows_v.at[bb], acc_sh.at[dst_v.at[jj]], ssem.at[bb],
                    add=True)

            return carry

        lax.fori_loop(0, S + GAHEAD, body, 0)
        for b in range(NBUF):
            pltpu.make_async_copy(
                rows_v.at[b], acc_sh.at[src_v.at[0]], ssem.at[b]).wait()
        plsc.subcore_barrier()
        pltpu.sync_copy(acc_sh.at[pl.ds(lo, CHUNK)],
                        a_out.at[cid, pl.ds(lo, CHUNK)])

    @functools.partial(
        pl.kernel,
        out_type=(jax.ShapeDtypeStruct((NC, NP, Hp), jnp.float32),
                  jax.ShapeDtypeStruct((NC, NP, Hp), jnp.float32)),
        mesh=mesh,
        compiler_params=sc_params,
        scratch_types=[
            pltpu.VMEM((S, B), jnp.int32),
            pltpu.VMEM((S, B), jnp.int32),
            pltpu.VMEM((16,), jnp.float32),
            pltpu.VMEM((CHUNK, Hp), jnp.float32),
            pltpu.VMEM((CHUNK, Hp), jnp.float32),
            pltpu.VMEM((CHUNK, Hp), jnp.float32),
            pltpu.VMEM((CHUNK, Hp), jnp.float32),
            pltpu.VMEM((NBUF, B, Hp), jnp.float32),
            pltpu.VMEM_SHARED((NP, Hp), jnp.float32),
            pltpu.SemaphoreType.DMA((NBUF,)),
            pltpu.SemaphoreType.DMA((NBUF,)),
        ],
    )
    def launch_b(src_hbm, dst_hbm, a_hbm, y_hbm, dinv_hbm, b1_hbm, z_hbm,
                 c_out, g_out,
                 src_v, dst_v, b1_v, dinv_v, p0_v, p1_v, y_v, rows_v,
                 acc_sh, gsem, ssem):
        cid = lax.axis_index("c")
        sid = lax.axis_index("s")
        lo = sid * CHUNK
        pltpu.sync_copy(src_hbm.at[2 * sid + cid], src_v)
        pltpu.sync_copy(dst_hbm.at[2 * sid + cid], dst_v)
        pltpu.sync_copy(b1_hbm, b1_v)
        pltpu.sync_copy(a_hbm.at[0, pl.ds(lo, CHUNK)], p0_v)
        pltpu.sync_copy(a_hbm.at[1, pl.ds(lo, CHUNK)], p1_v)
        pltpu.sync_copy(y_hbm.at[pl.ds(lo, CHUNK)], y_v)
        pltpu.sync_copy(dinv_hbm.at[pl.ds(lo, CHUNK)], dinv_v)
        pltpu.sync_copy(z_hbm.at[pl.ds(lo, CHUNK)],
                        acc_sh.at[pl.ds(lo, CHUNK)])
        b1row = b1_v[...]

        def gbody(i, carry):
            di = dinv_v[i, :]
            s = p0_v[i, :] + p1_v[i, :] + y_v[i, :]
            h = jnp.maximum(s * di + b1row, 0.0)
            p0_v[i, :] = h * di
            return carry

        lax.fori_loop(0, CHUNK, gbody, 0)
        pltpu.sync_copy(p0_v, g_out.at[cid, pl.ds(lo, CHUNK)])
        plsc.subcore_barrier()

        g2d = g_out.at[cid]

        def body(j, carry):
            @pl.when(j < S)
            def _gather():
                b = lax.rem(j, NBUF)

                @pl.when(j >= NBUF)
                def _wait_scatter():
                    pltpu.make_async_copy(
                        rows_v.at[b], acc_sh.at[src_v.at[0]], ssem.at[b]).wait()

                pltpu.async_copy(g2d.at[src_v.at[j]], rows_v.at[b],
                                 gsem.at[b])

            jj = j - GAHEAD

            @pl.when(jj >= 0)
            def _scatter():
                bb = lax.rem(jj, NBUF)
                pltpu.make_async_copy(
                    g2d.at[src_v.at[jj]], rows_v.at[bb], gsem.at[bb]).wait()
                pltpu.async_copy(
                    rows_v.at[bb], acc_sh.at[dst_v.at[jj]], ssem.at[bb],
                    add=True)

            return carry

        lax.fori_loop(0, S + GAHEAD, body, 0)
        for b in range(NBUF):
            pltpu.make_async_copy(
                rows_v.at[b], acc_sh.at[src_v.at[0]], ssem.at[b]).wait()
        plsc.subcore_barrier()
        pltpu.sync_copy(acc_sh.at[pl.ds(lo, CHUNK)],
                        c_out.at[cid, pl.ds(lo, CHUNK)])

    def prep_body(x_ref, w_ref, deg_ref, y_ref, dinv_ref):
        xw = jnp.dot(x_ref[...], w_ref[...],
                     preferred_element_type=jnp.float32)
        dt = deg_ref[0, pl.ds(0, N)] + deg_ref[1, pl.ds(0, N)]
        dinv = lax.rsqrt(dt + 1.0)[:, None]
        dinv_ref[pl.ds(0, N), :] = jnp.broadcast_to(dinv, (N, Hp))
        dinv_ref[pl.ds(N, NP - N), :] = jnp.ones((NP - N, Hp), jnp.float32)
        y_ref[pl.ds(0, N), :] = xw * dinv
        y_ref[pl.ds(N, NP - N), :] = jnp.zeros((NP - N, Hp), jnp.float32)

    prep = pl.pallas_call(
        prep_body,
        out_shape=(jax.ShapeDtypeStruct((NP, Hp), jnp.float32),
                   jax.ShapeDtypeStruct((NP, Hp), jnp.float32)))

    def fin_body(c0_ref, c1_ref, g_ref, dinv_ref, w2_ref, b2_ref, o_ref):
        t = dinv_ref[...] * (c0_ref[...] + c1_ref[...] + g_ref[...])
        o = jnp.dot(t, w2_ref[...],
                    preferred_element_type=jnp.float32) + b2_ref[...]
        col = lax.broadcasted_iota(jnp.int32, o.shape, 1)
        om = jnp.where(col < C, o, -jnp.inf)
        m = jnp.max(om, axis=1, keepdims=True)
        s = jnp.sum(jnp.where(col < C, jnp.exp(o - m), 0.0), axis=1,
                    keepdims=True)
        o_ref[...] = o - (m + jnp.log(s))

    fin = pl.pallas_call(
        fin_body, out_shape=jax.ShapeDtypeStruct((N, Hp), jnp.float32))

    return launch_h, launch_e, launch_b, prep, fin, NP, S, EP


def kernel(x, edge_index, W1, b1, W2, b2):
    N, D = x.shape
    E = edge_index.shape[1]
    H = W1.shape[1]
    C = W2.shape[1]
    Hp = 16
    (launch_h, launch_e, launch_b, prep, fin,
     NP, S, EP) = _build(N, E, D, H, C)

    src = edge_index[0].astype(jnp.int32)
    dst = edge_index[1].astype(jnp.int32)
    pad = EP - E
    lanes = jnp.arange(pad, dtype=jnp.int32) % 16
    srcp = jnp.concatenate([src, lanes]).reshape(NW, S, B)
    dstp = jnp.concatenate([dst, N + lanes]).reshape(NW, S, B)

    zr = jnp.zeros((NP, Hp), jnp.float32)
    W2p = jnp.concatenate(
        [W2, jnp.zeros((H, Hp - C), jnp.float32)], axis=1)
    b2p = jnp.concatenate([b2, jnp.zeros((Hp - C,), jnp.float32)]).reshape(1, Hp)

    deg = launch_h(dstp)
    y1, dinv2 = prep(x, W1, deg)
    a = launch_e(srcp, dstp, y1, zr)
    c, g = launch_b(srcp, dstp, a, y1, dinv2, b1, zr)
    out = fin(c[0, :N], c[1, :N], g[0, :N], dinv2[:N], W2p, b2p)
    return out[:, :C]


# speedup vs baseline: 1.2080x; 1.0181x over previous
import functools

import jax
import jax.numpy as jnp
from jax import lax
from jax.experimental import pallas as pl
from jax.experimental.pallas import tpu as pltpu
from jax.experimental.pallas import tpu_sc as plsc

NC = 2
NS = 16
NW = NC * NS
B = 128
NBUF = 8
GAHEAD = 4


@functools.cache
def _build(N, E, D, H, C):
    assert H == 16, "row width must match the 64B DMA granule"
    Hp = 16
    NP = -(-(N + 16) // 256) * 256
    CHUNK = NP // NS
    S = -(-E // (NW * B))
    S = max(S + (S % 2), NBUF)
    EP = S * NW * B

    mesh = plsc.VectorSubcoreMesh(core_axis_name="c", subcore_axis_name="s")
    sc_params = pltpu.CompilerParams(use_tc_tiling_on_sc=False)

    @functools.partial(
        pl.kernel,
        out_type=jax.ShapeDtypeStruct((NC, NP), jnp.float32),
        mesh=mesh,
        compiler_params=sc_params,
        scratch_types=[
            pltpu.VMEM((S, B), jnp.int32),
            pltpu.VMEM((B,), jnp.float32),
            pltpu.VMEM((CHUNK,), jnp.float32),
            pltpu.VMEM_SHARED((NP,), jnp.float32),
            pltpu.SemaphoreType.DMA((NBUF,)),
        ],
    )
    def launch_h(dst_hbm, deg_out, dst_v, ones_v, z_v, deg_sh, ssem):
        cid = lax.axis_index("c")
        sid = lax.axis_index("s")
        lo = sid * CHUNK
        pltpu.sync_copy(dst_hbm.at[2 * sid + cid], dst_v)
        for i in range(B // 16):
            ones_v[pl.ds(i * 16, 16)] = jnp.ones((16,), jnp.float32)

        def zbody(i, carry):
            z_v[pl.ds(i * 16, 16)] = jnp.zeros((16,), jnp.float32)
            return carry

        lax.fori_loop(0, CHUNK // 16, zbody, 0)
        pltpu.sync_copy(z_v, deg_sh.at[pl.ds(lo, CHUNK)])
        plsc.subcore_barrier()

        def dbody(j, carry):
            b = lax.rem(j, NBUF)

            @pl.when(j >= NBUF)
            def _wait_prev():
                pltpu.make_async_copy(
                    ones_v, deg_sh.at[dst_v.at[0]], ssem.at[b]).wait()

            pltpu.async_copy(ones_v, deg_sh.at[dst_v.at[j]], ssem.at[b],
                             add=True)
            return carry

        lax.fori_loop(0, S, dbody, 0)
        for b in range(NBUF):
            pltpu.make_async_copy(
                ones_v, deg_sh.at[dst_v.at[0]], ssem.at[b]).wait()
        plsc.subcore_barrier()
        pltpu.sync_copy(deg_sh.at[pl.ds(lo, CHUNK)],
                        deg_out.at[cid, pl.ds(lo, CHUNK)])

    @functools.partial(
        pl.kernel,
        out_type=jax.ShapeDtypeStruct((NC, NP, Hp), jnp.float32),
        mesh=mesh,
        compiler_params=sc_params,
        scratch_types=[
            pltpu.VMEM((S, B), jnp.int32),
            pltpu.VMEM((S, B), jnp.int32),
            pltpu.VMEM((NBUF, B, Hp), jnp.float32),
            pltpu.VMEM_SHARED((NP, Hp), jnp.float32),
            pltpu.SemaphoreType.DMA((NBUF,)),
            pltpu.SemaphoreType.DMA((NBUF,)),
        ],
    )
    def launch_e(src_hbm, dst_hbm, y_hbm, z_hbm,
                 a_out, src_v, dst_v, rows_v, acc_sh, gsem, ssem):
        cid = lax.axis_index("c")
        sid = lax.axis_index("s")
        lo = sid * CHUNK
        pltpu.sync_copy(src_hbm.at[2 * sid + cid], src_v)
        pltpu.sync_copy(dst_hbm.at[2 * sid + cid], dst_v)
        pltpu.sync_copy(z_hbm.at[pl.ds(lo, CHUNK)],
                        acc_sh.at[pl.ds(lo, CHUNK)])
        plsc.subcore_barrier()

        def body(j, carry):
            @pl.when(j < S)
            def _gather():
                b = lax.rem(j, NBUF)

                @pl.when(j >= NBUF)
                def _wait_scatter():
                    pltpu.make_async_copy(
                        rows_v.at[b], acc_sh.at[src_v.at[0]], ssem.at[b]).wait()

                pltpu.async_copy(y_hbm.at[src_v.at[j]], rows_v.at[b],
                                 gsem.at[b])

            jj = j - GAHEAD

            @pl.when(jj >= 0)
            def _scatter():
                bb = lax.rem(jj, NBUF)
                pltpu.make_async_copy(
                    y_hbm.at[src_v.at[jj]], rows_v.at[bb], gsem.at[bb]).wait()
                pltpu.async_copy(
                    rows_v.at[bb], acc_sh.at[dst_v.at[jj]], ssem.at[bb],
                    add=True)

            return carry

        lax.fori_loop(0, S + GAHEAD, body, 0)
        for b in range(NBUF):
            pltpu.make_async_copy(
                rows_v.at[b], acc_sh.at[src_v.at[0]], ssem.at[b]).wait()
        plsc.subcore_barrier()
        pltpu.sync_copy(acc_sh.at[pl.ds(lo, CHUNK)],
                        a_out.at[cid, pl.ds(lo, CHUNK)])

    @functools.partial(
        pl.kernel,
        out_type=(jax.ShapeDtypeStruct((NC, NP, Hp), jnp.float32),
                  jax.ShapeDtypeStruct((NC, NP, Hp), jnp.float32)),
        mesh=mesh,
        compiler_params=sc_params,
        scratch_types=[
            pltpu.VMEM((S, B), jnp.int32),
            pltpu.VMEM((S, B), jnp.int32),
            pltpu.VMEM((16,), jnp.float32),
            pltpu.VMEM((CHUNK, Hp), jnp.float32),
            pltpu.VMEM((CHUNK, Hp), jnp.float32),
            pltpu.VMEM((CHUNK, Hp), jnp.float32),
            pltpu.VMEM((CHUNK, Hp), jnp.float32),
            pltpu.VMEM((NBUF, B, Hp), jnp.float32),
            pltpu.VMEM_SHARED((NP, Hp), jnp.float32),
            pltpu.SemaphoreType.DMA((NBUF,)),
            pltpu.SemaphoreType.DMA((NBUF,)),
        ],
    )
    def launch_b(src_hbm, dst_hbm, a_hbm, y_hbm, dinv_hbm, b1_hbm, z_hbm,
                 c_out, g_out,
                 src_v, dst_v, b1_v, dinv_v, p0_v, p1_v, y_v, rows_v,
                 acc_sh, gsem, ssem):
        cid = lax.axis_index("c")
        sid = lax.axis_index("s")
        lo = sid * CHUNK
        pltpu.sync_copy(src_hbm.at[2 * sid + cid], src_v)
        pltpu.sync_copy(dst_hbm.at[2 * sid + cid], dst_v)
        pltpu.sync_copy(b1_hbm, b1_v)
        pltpu.sync_copy(a_hbm.at[0, pl.ds(lo, CHUNK)], p0_v)
        pltpu.sync_copy(a_hbm.at[1, pl.ds(lo, CHUNK)], p1_v)
        pltpu.sync_copy(y_hbm.at[pl.ds(lo, CHUNK)], y_v)
        pltpu.sync_copy(dinv_hbm.at[pl.ds(lo, CHUNK)], dinv_v)
        pltpu.sync_copy(z_hbm.at[pl.ds(lo, CHUNK)],
                        acc_sh.at[pl.ds(lo, CHUNK)])
        b1row = b1_v[...]

        def gbody(i, carry):
            di = dinv_v[i, :]
            s = p0_v[i, :] + p1_v[i, :] + y_v[i, :]
            h = jnp.maximum(s * di + b1row, 0.0)
            p0_v[i, :] = h * di
            return carry

        lax.fori_loop(0, CHUNK, gbody, 0)
        pltpu.sync_copy(p0_v, g_out.at[cid, pl.ds(lo, CHUNK)])
        plsc.subcore_barrier()

        g2d = g_out.at[cid]

        def body(j, carry):
            @pl.when(j < S)
            def _gather():
                b = lax.rem(j, NBUF)

                @pl.when(j >= NBUF)
                def _wait_scatter():
                    pltpu.make_async_copy(
                        rows_v.at[b], acc_sh.at[src_v.at[0]], ssem.at[b]).wait()

                pltpu.async_copy(g2d.at[src_v.at[j]], rows_v.at[b],
                                 gsem.at[b])

            jj = j - GAHEAD

            @pl.when(jj >= 0)
            def _scatter():
                bb = lax.rem(jj, NBUF)
                pltpu.make_async_copy(
                    g2d.at[src_v.at[jj]], rows_v.at[bb], gsem.at[bb]).wait()
                pltpu.async_copy(
                    rows_v.at[bb], acc_sh.at[dst_v.at[jj]], ssem.at[bb],
                    add=True)

            return carry

        lax.fori_loop(0, S + GAHEAD, body, 0)
        for b in range(NBUF):
            pltpu.make_async_copy(
                rows_v.at[b], acc_sh.at[src_v.at[0]], ssem.at[b]).wait()
        plsc.subcore_barrier()
        pltpu.sync_copy(acc_sh.at[pl.ds(lo, CHUNK)],
                        c_out.at[cid, pl.ds(lo, CHUNK)])

    def mm_body(x_ref, w_ref, o_ref):
        o_ref[...] = jnp.dot(x_ref[...], w_ref[...],
                             preferred_element_type=jnp.float32)

    mm1 = pl.pallas_call(
        mm_body, out_shape=jax.ShapeDtypeStruct((N, Hp), jnp.float32))

    def prep_body(xw_ref, deg_ref, y_ref, dinv_ref):
        dt = deg_ref[0, pl.ds(0, N)] + deg_ref[1, pl.ds(0, N)]
        dinv = lax.rsqrt(dt + 1.0)[:, None]
        dinv_ref[pl.ds(0, N), :] = jnp.broadcast_to(dinv, (N, Hp))
        dinv_ref[pl.ds(N, NP - N), :] = jnp.ones((NP - N, Hp), jnp.float32)
        y_ref[pl.ds(0, N), :] = xw_ref[...] * dinv
        y_ref[pl.ds(N, NP - N), :] = jnp.zeros((NP - N, Hp), jnp.float32)

    prep = pl.pallas_call(
        prep_body,
        out_shape=(jax.ShapeDtypeStruct((NP, Hp), jnp.float32),
                   jax.ShapeDtypeStruct((NP, Hp), jnp.float32)))

    def fin_body(c_ref, g_ref, dinv_ref, w2_ref, b2_ref, o_ref):
        t = dinv_ref[pl.ds(0, N), :] * (
            c_ref[0, pl.ds(0, N), :] + c_ref[1, pl.ds(0, N), :]
            + g_ref[0, pl.ds(0, N), :])
        o = jnp.dot(t, w2_ref[...],
                    preferred_element_type=jnp.float32) + b2_ref[...]
        col = lax.broadcasted_iota(jnp.int32, o.shape, 1)
        om = jnp.where(col < C, o, -jnp.inf)
        m = jnp.max(om, axis=1, keepdims=True)
        s = jnp.sum(jnp.where(col < C, jnp.exp(o - m), 0.0), axis=1,
                    keepdims=True)
        o_ref[...] = o - (m + jnp.log(s))

    fin = pl.pallas_call(
        fin_body, out_shape=jax.ShapeDtypeStruct((N, Hp), jnp.float32))

    return launch_h, launch_e, launch_b, mm1, prep, fin, NP, S, EP


def kernel(x, edge_index, W1, b1, W2, b2):
    N, D = x.shape
    E = edge_index.shape[1]
    H = W1.shape[1]
    C = W2.shape[1]
    Hp = 16
    (launch_h, launch_e, launch_b, mm1, prep, fin,
     NP, S, EP) = _build(N, E, D, H, C)

    src = edge_index[0].astype(jnp.int32)
    dst = edge_index[1].astype(jnp.int32)
    pad = EP - E
    lanes = jnp.arange(pad, dtype=jnp.int32) % 16
    srcp = jnp.concatenate([src, lanes]).reshape(NW, S, B)
    dstp = jnp.concatenate([dst, N + lanes]).reshape(NW, S, B)

    zr = jnp.zeros((NP, Hp), jnp.float32)
    W2p = jnp.concatenate(
        [W2, jnp.zeros((H, Hp - C), jnp.float32)], axis=1)
    b2p = jnp.concatenate([b2, jnp.zeros((Hp - C,), jnp.float32)]).reshape(1, Hp)

    xw = mm1(x, W1)
    deg = launch_h(dstp)
    y1, dinv2 = prep(xw, deg)
    a = launch_e(srcp, dstp, y1, zr)
    c, g = launch_b(srcp, dstp, a, y1, dinv2, b1, zr)
    out = fin(c, g, dinv2, W2p, b2p)
    return out[:, :C]
